# baseline (speedup 1.0000x reference)
import functools

import jax
import jax.numpy as jnp
from jax import lax
from jax.experimental import pallas as pl
from jax.experimental.pallas import tpu as pltpu
from jax.experimental.pallas import tpu_sc as plsc

VOCAB = 2000
EMBED_DIM = 19
TFLAT = VOCAB * EMBED_DIM
N_IDX = 16384 * 50
IDX_COLS = 128
IDX_ROWS = N_IDX // IDX_COLS
NS = 16
ROWS_S = IDX_ROWS // NS
RPS = 128
SLICE = RPS * EMBED_DIM
MAX_START = TFLAT - SLICE
GROUPS = IDX_COLS // 16
INV_N = 1.0 / float(N_IDX * EMBED_DIM)


def _sc_embedding_mean_sigmoid(idx2d, table_flat):
    mesh = plsc.VectorSubcoreMesh(
        core_axis_name="c", subcore_axis_name="s", num_cores=1
    )

    @functools.partial(
        pl.kernel,
        mesh=mesh,
        compiler_params=pltpu.CompilerParams(needs_layout_passes=False),
        out_type=jax.ShapeDtypeStruct((16,), jnp.float32),
        scratch_types=[
            pltpu.VMEM((ROWS_S, IDX_COLS), jnp.int32),
            pltpu.VMEM((SLICE,), jnp.float32),
            pltpu.VMEM((RPS,), jnp.float32),
            pltpu.VMEM((NS * RPS,), jnp.float32),
            pltpu.VMEM((16,), jnp.float32),
            pltpu.VMEM((NS, 16), jnp.float32),
            pltpu.VMEM_SHARED((NS * RPS,), jnp.float32),
            pltpu.VMEM_SHARED((NS, 16), jnp.float32),
            pltpu.SemaphoreType.DMA,
        ],
    )
    def k(idx_hbm, tab_hbm, out_hbm, idx_v, tab_v, rs_local, rs_v, stage_v,
          part_v, sh_rs, sh_part, sem):
        sid = lax.axis_index("s")
        idx_cp = pltpu.async_copy(
            idx_hbm.at[pl.ds(sid * ROWS_S, ROWS_S), :], idx_v, sem
        )

        start = pl.multiple_of(jnp.minimum(sid * SLICE, MAX_START), 8)
        pltpu.sync_copy(tab_hbm.at[pl.ds(start, SLICE)], tab_v)
        lanes = lax.iota(jnp.int32, 16)
        for ch in range(RPS // 16):
            r = sid * RPS + ch * 16 + lanes
            rc = jnp.minimum(r, VOCAB - 1)
            base = rc * EMBED_DIM - start
            acc = plsc.load_gather(tab_v, [base])
            for d in range(1, EMBED_DIM):
                acc = acc + plsc.load_gather(tab_v, [base + d])
            acc = jnp.where(r < VOCAB, acc, 0.0)
            rs_local[pl.ds(ch * 16, 16)] = acc
        pltpu.sync_copy(rs_local, sh_rs.at[pl.ds(sid * RPS, RPS)])
        plsc.subcore_barrier()
        pltpu.sync_copy(sh_rs, rs_v)
        idx_cp.wait()

        def body(i, carry):
            out = []
            for u in range(GROUPS):
                iv = idx_v[i, pl.ds(u * 16, 16)]
                out.append(carry[u] + plsc.load_gather(rs_v, [iv]))
            return tuple(out)

        accs = lax.fori_loop(
            0, ROWS_S, body,
            tuple(jnp.zeros((16,), jnp.float32) for _ in range(GROUPS)),
        )
        tot = accs[0]
        for u in range(1, GROUPS):
            tot = tot + accs[u]
        stage_v[...] = tot
        pltpu.sync_copy(stage_v, sh_part.at[sid])
        plsc.subcore_barrier()

        @pl.when(sid == 0)
        def _():
            pltpu.sync_copy(sh_part, part_v)
            tv = part_v[0]
            for i in range(1, NS):
                tv = tv + part_v[i]
            s = jnp.sum(tv) * INV_N
            vec = jnp.broadcast_to(s, (16,))
            stage_v[...] = 1.0 / (1.0 + jnp.exp(-vec))
            pltpu.sync_copy(stage_v, out_hbm)

    return k(idx2d, table_flat)


def kernel(movies, table):
    out = _sc_embedding_mean_sigmoid(
        movies.reshape(IDX_ROWS, IDX_COLS), table.reshape(TFLAT)
    )
    return out[0]


# speedup vs baseline: 1.1967x; 1.1967x over previous
import functools

import jax
import jax.numpy as jnp
from jax import lax
from jax.experimental import pallas as pl
from jax.experimental.pallas import tpu as pltpu
from jax.experimental.pallas import tpu_sc as plsc

VOCAB = 2000
EMBED_DIM = 19
N_ROWS = 16384
N_COLS = 50
N_IDX = N_ROWS * N_COLS
NS = 16
ROWS_S = N_ROWS // NS
CH_ROWS = 256
N_CH = ROWS_S // CH_ROWS
RPS = 128
MAX_R0 = VOCAB - RPS
TAIL0 = N_COLS - 16
TAIL_NEW = 16 - (N_COLS - 3 * 16)
INV_N = 1.0 / float(N_IDX * EMBED_DIM)


def _sc_embedding_mean_sigmoid(movies, table):
    mesh = plsc.VectorSubcoreMesh(
        core_axis_name="c", subcore_axis_name="s", num_cores=1
    )

    @functools.partial(
        pl.kernel,
        mesh=mesh,
        compiler_params=pltpu.CompilerParams(needs_layout_passes=False),
        out_type=jax.ShapeDtypeStruct((16,), jnp.float32),
        scratch_types=[
            pltpu.VMEM((2, CH_ROWS, N_COLS), jnp.int32),
            pltpu.VMEM((RPS, EMBED_DIM), jnp.float32),
            pltpu.VMEM((RPS,), jnp.float32),
            pltpu.VMEM((NS * RPS,), jnp.float32),
            pltpu.VMEM((16,), jnp.float32),
            pltpu.VMEM((NS, 16), jnp.float32),
            pltpu.VMEM_SHARED((NS * RPS,), jnp.float32),
            pltpu.VMEM_SHARED((NS, 16), jnp.float32),
            pltpu.SemaphoreType.DMA,
            pltpu.SemaphoreType.DMA,
        ],
    )
    def k(idx_hbm, tab_hbm, out_hbm, idx_v, tab_v, rs_local, rs_v, stage_v,
          part_v, sh_rs, sh_part, sem0, sem1):
        sid = lax.axis_index("s")
        sems = (sem0, sem1)
        row0 = sid * ROWS_S
        cps = [None] * N_CH
        cps[0] = pltpu.async_copy(
            idx_hbm.at[pl.ds(row0, CH_ROWS), :], idx_v.at[0], sems[0]
        )

        r0 = jnp.minimum(sid * RPS, MAX_R0)
        pltpu.sync_copy(tab_hbm.at[pl.ds(r0, RPS), :], tab_v)
        lanes = lax.iota(jnp.int32, 16)
        for ch in range(RPS // 16):
            r = sid * RPS + ch * 16 + lanes
            lr = jnp.minimum(r, VOCAB - 1) - r0
            acc = plsc.load_gather(tab_v, [lr, jnp.zeros((16,), jnp.int32)])
            for d in range(1, EMBED_DIM):
                acc = acc + plsc.load_gather(
                    tab_v, [lr, jnp.full((16,), d, jnp.int32)]
                )
            acc = jnp.where(r < VOCAB, acc, 0.0)
            rs_local[pl.ds(ch * 16, 16)] = acc
        pltpu.sync_copy(rs_local, sh_rs.at[pl.ds(sid * RPS, RPS)])
        plsc.subcore_barrier()
        pltpu.sync_copy(sh_rs, rs_v)

        tail_mask = lanes >= TAIL_NEW
        accs = tuple(jnp.zeros((16,), jnp.float32) for _ in range(4))

        for ch in range(N_CH):
            cur = ch % 2
            if ch + 1 < N_CH:
                cps[ch + 1] = pltpu.async_copy(
                    idx_hbm.at[pl.ds(row0 + (ch + 1) * CH_ROWS, CH_ROWS), :],
                    idx_v.at[(ch + 1) % 2],
                    sems[(ch + 1) % 2],
                )
            cps[ch].wait()
            buf = idx_v.at[cur]

            def body(r, carry):
                a0, a1, a2, a3 = carry
                iv0 = buf[r, pl.ds(0, 16)]
                iv1 = buf[r, pl.ds(16, 16)]
                iv2 = buf[r, pl.ds(32, 16)]
                iv3 = buf[r, pl.ds(TAIL0, 16)]
                a0 = a0 + plsc.load_gather(rs_v, [iv0])
                a1 = a1 + plsc.load_gather(rs_v, [iv1])
                a2 = a2 + plsc.load_gather(rs_v, [iv2])
                a3 = a3 + jnp.where(
                    tail_mask, plsc.load_gather(rs_v, [iv3]), 0.0
                )
                return a0, a1, a2, a3

            accs = lax.fori_loop(0, CH_ROWS, body, accs)

        tot = accs[0] + accs[1] + accs[2] + accs[3]
        stage_v[...] = tot
        pltpu.sync_copy(stage_v, sh_part.at[sid])
        plsc.subcore_barrier()

        @pl.when(sid == 0)
        def _():
            pltpu.sync_copy(sh_part, part_v)
            tv = part_v[0]
            for i in range(1, NS):
                tv = tv + part_v[i]
            s = jnp.sum(tv) * INV_N
            vec = jnp.broadcast_to(s, (16,))
            stage_v[...] = 1.0 / (1.0 + jnp.exp(-vec))
            pltpu.sync_copy(stage_v, out_hbm)

    return k(movies, table)


def kernel(movies, table):
    out = _sc_embedding_mean_sigmoid(movies, table)
    return out[0]


# speedup vs baseline: 1.4450x; 1.2075x over previous
import functools

import jax
import jax.numpy as jnp
from jax import lax
from jax.experimental import pallas as pl
from jax.experimental.pallas import tpu as pltpu
from jax.experimental.pallas import tpu_sc as plsc

VOCAB = 2000
EMBED_DIM = 19
TFLAT = VOCAB * EMBED_DIM
N_IDX = 16384 * 50
NC = 2
NS = 16
NW = NC * NS
PER_W = N_IDX // NW
CHUNK = PER_W // 2
RPS = 128
UNROLL = 8
STEPS = CHUNK // (16 * UNROLL)
INV_N = 1.0 / float(N_IDX * EMBED_DIM)


def _finalize_body(part_ref, out_ref):
    total = jnp.sum(part_ref[...], axis=(0, 1), keepdims=True)
    out_ref[...] = jax.nn.sigmoid(total * INV_N)


def _sc_gather_sum(idx_flat, tab_flat):
    mesh = plsc.VectorSubcoreMesh(core_axis_name="c", subcore_axis_name="s")

    @functools.partial(
        pl.kernel,
        mesh=mesh,
        compiler_params=pltpu.CompilerParams(needs_layout_passes=False),
        out_type=jax.ShapeDtypeStruct((NC, 16), jnp.float32),
        scratch_types=[
            pltpu.VMEM((CHUNK,), jnp.int32),
            pltpu.VMEM((CHUNK,), jnp.int32),
            pltpu.VMEM((EMBED_DIM * RPS,), jnp.float32),
            pltpu.VMEM((RPS,), jnp.float32),
            pltpu.VMEM((NS * RPS,), jnp.float32),
            pltpu.VMEM((16,), jnp.float32),
            pltpu.VMEM((NS, 16), jnp.float32),
            pltpu.VMEM_SHARED((NS * RPS,), jnp.float32),
            pltpu.VMEM_SHARED((NS, 16), jnp.float32),
            pltpu.SemaphoreType.DMA,
            pltpu.SemaphoreType.DMA,
            pltpu.SemaphoreType.DMA,
        ],
    )
    def k(idx_hbm, tab_hbm, out_hbm, idx_a, idx_b, tab_v, rs_local, rs_v,
          stage_v, part_v, sh_rs, sh_part, sem0, sem1, semt):
        cid = lax.axis_index("c")
        sid = lax.axis_index("s")
        wid = sid * NC + cid
        base = wid * PER_W
        bufs = (idx_a, idx_b)
        sems = (sem0, sem1)
        cps = [None, None]
        cps[0] = pltpu.async_copy(
            idx_hbm.at[pl.ds(base, CHUNK)], idx_a, sems[0]
        )

        c0 = sid * RPS
        tcps = [
            pltpu.async_copy(
                tab_hbm.at[pl.ds(d * VOCAB + c0, RPS)],
                tab_v.at[pl.ds(d * RPS, RPS)],
                semt,
            )
            for d in range(EMBED_DIM)
        ]
        for cp in tcps:
            cp.wait()
        for g in range(RPS // 16):
            acc = tab_v[pl.ds(g * 16, 16)]
            for d in range(1, EMBED_DIM):
                acc = acc + tab_v[pl.ds(d * RPS + g * 16, 16)]
            rs_local[pl.ds(g * 16, 16)] = acc
        pltpu.sync_copy(rs_local, sh_rs.at[pl.ds(sid * RPS, RPS)])
        plsc.subcore_barrier()
        pltpu.sync_copy(sh_rs, rs_v)

        accs = tuple(jnp.zeros((16,), jnp.float32) for _ in range(UNROLL))
        for ch in range(2):
            if ch == 0:
                cps[1] = pltpu.async_copy(
                    idx_hbm.at[pl.ds(base + CHUNK, CHUNK)], idx_b, sems[1]
                )
            cps[ch].wait()
            buf = bufs[ch]

            def body(i, carry):
                out = []
                for u in range(UNROLL):
                    iv = buf[pl.ds((i * UNROLL + u) * 16, 16)]
                    out.append(carry[u] + plsc.load_gather(rs_v, [iv]))
                return tuple(out)

            accs = lax.fori_loop(0, STEPS, body, accs)

        tot = accs[0]
        for u in range(1, UNROLL):
            tot = tot + accs[u]
        stage_v[...] = tot
        pltpu.sync_copy(stage_v, sh_part.at[sid])
        plsc.subcore_barrier()

        @pl.when(sid == 0)
        def _():
            pltpu.sync_copy(sh_part, part_v)
            tv = part_v[0]
            for i in range(1, NS):
                tv = tv + part_v[i]
            stage_v[...] = tv
            pltpu.sync_copy(stage_v, out_hbm.at[cid])

    return k(idx_flat, tab_flat)


def kernel(movies, table):
    partials = _sc_gather_sum(
        movies.T.reshape(N_IDX), table.T.reshape(TFLAT)
    )
    out = pl.pallas_call(
        _finalize_body,
        out_shape=jax.ShapeDtypeStruct((1, 1), jnp.float32),
    )(partials)
    return out.reshape(())
